# baseline (speedup 1.0000x reference)
import jax
import jax.numpy as jnp
from jax import lax
from jax.experimental import pallas as pl
from jax.experimental.pallas import tpu as pltpu
from jax.experimental.pallas import tpu_sc as plsc

N_EMB = 1000000
EMB_DIM = 32
EMB_SCALE = 10.0

N_B = 16384
N_S = 50
LANES = 16
BT = 128
N_BT = N_B // BT
NUM_WORKERS = 32
BT_PER_W = N_BT // NUM_WORKERS
XBLK = BT * N_S
NGRP = BT // LANES
SB = 5
GB = SB * BT
NBATCH = N_S // SB


def _sc_kernel(x_hbm, table_hbm, out_hbm,
               x_blk, idx_sm, rows_a, rows_b, out_a, out_b,
               sem_ga, sem_gb, sem_oa, sem_ob):
    cid = lax.axis_index("c")
    sid = lax.axis_index("s")
    wid = sid * 2 + cid

    lane = lax.iota(jnp.int32, LANES)
    lane50 = lane * N_S

    def fire(k, rows_v, sem):
        return pltpu.async_copy(
            table_hbm.at[idx_sm.at[pl.ds(k * GB, GB)]], rows_v, sem)

    def wait_g(k, rows_v, sem):
        pltpu.make_async_copy(
            table_hbm.at[idx_sm.at[pl.ds(k * GB, GB)]], rows_v, sem).wait()

    @pl.loop(0, BT_PER_W)
    def _bt(j):
        tc = wid * BT_PER_W + j
        pltpu.sync_copy(x_hbm.at[pl.ds(tc * XBLK, XBLK)], x_blk)

        @pl.loop(0, N_S)
        def _ex(s):
            for g in range(NGRP):
                xv = plsc.load_gather(x_blk, [lane50 + (g * LANES * N_S + s)])
                idx_sm[pl.ds(s * BT + g * LANES, LANES)] = xv

        def process(k, q, rows_v, sem_g, out_t, sem_o):
            wait_g(k, rows_v, sem_g)

            @pl.loop(0, SB)
            def _s(ss):
                @pl.when((q > 0) | (ss > 0))
                def _dr():
                    pltpu.make_async_copy(
                        out_t, out_hbm.at[0, :, tc], sem_o).wait()

                roff = ss * BT
                for g in range(NGRP):
                    ridx = lane + (roff + g * LANES)
                    for d in range(EMB_DIM):
                        v = plsc.load_gather(
                            rows_v, [ridx, jnp.full((LANES,), d, jnp.int32)])
                        out_t[d // 8, d % 8, pl.ds(g * LANES, LANES)] = \
                            v * EMB_SCALE
                pltpu.async_copy(out_t, out_hbm.at[k * SB + ss, :, tc], sem_o)

        fire(0, rows_a, sem_ga)

        @pl.loop(0, NBATCH // 2)
        def _q(q):
            k0 = 2 * q
            fire(k0 + 1, rows_b, sem_gb)
            process(k0, q, rows_a, sem_ga, out_a, sem_oa)

            @pl.when(k0 + 2 < NBATCH)
            def _nx():
                fire(k0 + 2, rows_a, sem_ga)

            process(k0 + 1, q, rows_b, sem_gb, out_b, sem_ob)

        pltpu.make_async_copy(out_a, out_hbm.at[0, :, tc], sem_oa).wait()
        pltpu.make_async_copy(out_b, out_hbm.at[0, :, tc], sem_ob).wait()


@jax.jit
def _scaled_embedding(x2d, table):
    mesh = plsc.VectorSubcoreMesh(core_axis_name="c", subcore_axis_name="s")
    out5 = pl.kernel(
        _sc_kernel,
        out_type=jax.ShapeDtypeStruct((N_S, 4, N_BT, 8, BT), jnp.float32),
        mesh=mesh,
        scratch_types=[
            pltpu.VMEM((XBLK,), jnp.int32),
            pltpu.VMEM((XBLK,), jnp.int32),
            pltpu.VMEM((GB, EMB_DIM), jnp.float32),
            pltpu.VMEM((GB, EMB_DIM), jnp.float32),
            pltpu.VMEM((4, 8, BT), jnp.float32),
            pltpu.VMEM((4, 8, BT), jnp.float32),
            pltpu.SemaphoreType.DMA,
            pltpu.SemaphoreType.DMA,
            pltpu.SemaphoreType.DMA,
            pltpu.SemaphoreType.DMA,
        ],
        compiler_params=pltpu.CompilerParams(
            use_tc_tiling_on_sc=False, needs_layout_passes=False
        ),
    )(x2d.reshape(-1), table)
    return out5.transpose((2, 4, 0, 1, 3)).reshape(N_B, N_S, EMB_DIM)


def kernel(x, table):
    return _scaled_embedding(x, table)


# speedup vs baseline: 1.1041x; 1.1041x over previous
import jax
import jax.numpy as jnp
from jax import lax
from jax.experimental import pallas as pl
from jax.experimental.pallas import tpu as pltpu
from jax.experimental.pallas import tpu_sc as plsc

N_EMB = 1000000
EMB_DIM = 32
EMB_SCALE = 10.0

N_B = 16384
N_S = 50
LANES = 16
BT = 128
N_BT = N_B // BT
NUM_WORKERS = 32
BT_PER_W = N_BT // NUM_WORKERS
XBLK = BT * N_S
NGRP = BT // LANES
SB = 5
GB = SB * BT
NBATCH = N_S // SB
OUTW = 4 * 8 * BT


def _sc_kernel(x_hbm, table_hbm, out_hbm,
               x_blk, idx_sm, rows_a, rows_b, out_a, out_b,
               sem_ga, sem_gb, sem_oa, sem_ob):
    cid = lax.axis_index("c")
    sid = lax.axis_index("s")
    wid = sid * 2 + cid

    lane = lax.iota(jnp.int32, LANES)
    lane50 = lane * N_S
    lane128 = lane * BT

    def fire(k, rows_v, sem):
        return pltpu.async_copy(
            table_hbm.at[idx_sm.at[pl.ds(k * GB, GB)]], rows_v, sem)

    def wait_g(k, rows_v, sem):
        pltpu.make_async_copy(
            table_hbm.at[idx_sm.at[pl.ds(k * GB, GB)]], rows_v, sem).wait()

    @pl.loop(0, BT_PER_W)
    def _bt(j):
        tc = wid * BT_PER_W + j
        pltpu.sync_copy(x_hbm.at[pl.ds(tc * XBLK, XBLK)], x_blk)

        @pl.loop(0, N_S)
        def _ex(s):
            for g in range(NGRP):
                xv = plsc.load_gather(x_blk, [lane50 + (g * LANES * N_S + s)])
                idx_sm[pl.ds(s * BT + g * LANES, LANES)] = xv

        def drain_out(out_t, sem_o):
            for tr in range(4):
                pltpu.make_async_copy(
                    out_t.at[pl.ds(tr * 1024, 1024)],
                    out_hbm.at[0, tr, tc], sem_o).wait()

        def process(k, q, rows_v, sem_g, out_t, sem_o):
            wait_g(k, rows_v, sem_g)

            @pl.loop(0, SB)
            def _s(ss):
                @pl.when((q > 0) | (ss > 0))
                def _dr():
                    drain_out(out_t, sem_o)

                roff = ss * BT

                @pl.loop(0, BT, unroll=8)
                def _b(b):
                    for h in range(EMB_DIM // LANES):
                        v = rows_v[roff + b, pl.ds(h * LANES, LANES)]
                        plsc.store_scatter(
                            out_t, [lane128 + (h * LANES * BT + b)],
                            v * EMB_SCALE)

                s_glob = k * SB + ss
                for tr in range(4):
                    pltpu.async_copy(
                        out_t.at[pl.ds(tr * 1024, 1024)],
                        out_hbm.at[s_glob, tr, tc], sem_o)

        fire(0, rows_a, sem_ga)

        @pl.loop(0, NBATCH // 2)
        def _q(q):
            k0 = 2 * q
            fire(k0 + 1, rows_b, sem_gb)
            process(k0, q, rows_a, sem_ga, out_a, sem_oa)

            @pl.when(k0 + 2 < NBATCH)
            def _nx():
                fire(k0 + 2, rows_a, sem_ga)

            process(k0 + 1, q, rows_b, sem_gb, out_b, sem_ob)

        drain_out(out_a, sem_oa)
        drain_out(out_b, sem_ob)


@jax.jit
def _scaled_embedding(x2d, table):
    mesh = plsc.VectorSubcoreMesh(core_axis_name="c", subcore_axis_name="s")
    out6 = pl.kernel(
        _sc_kernel,
        out_type=jax.ShapeDtypeStruct((N_S, 4, N_BT, 8 * BT), jnp.float32),
        mesh=mesh,
        scratch_types=[
            pltpu.VMEM((XBLK,), jnp.int32),
            pltpu.VMEM((XBLK,), jnp.int32),
            pltpu.VMEM((GB, EMB_DIM), jnp.float32),
            pltpu.VMEM((GB, EMB_DIM), jnp.float32),
            pltpu.VMEM((OUTW,), jnp.float32),
            pltpu.VMEM((OUTW,), jnp.float32),
            pltpu.SemaphoreType.DMA,
            pltpu.SemaphoreType.DMA,
            pltpu.SemaphoreType.DMA,
            pltpu.SemaphoreType.DMA,
        ],
        compiler_params=pltpu.CompilerParams(
            use_tc_tiling_on_sc=False, needs_layout_passes=False
        ),
    )(x2d.reshape(-1), table)
    out5 = out6.reshape(N_S, 4, N_BT, 8, BT)
    return out5.transpose((2, 4, 0, 1, 3)).reshape(N_B, N_S, EMB_DIM)


def kernel(x, table):
    return _scaled_embedding(x, table)


# speedup vs baseline: 1.3530x; 1.2255x over previous
import jax
import jax.numpy as jnp
from jax import lax
from jax.experimental import pallas as pl
from jax.experimental.pallas import tpu as pltpu
from jax.experimental.pallas import tpu_sc as plsc

N_EMB = 1000000
EMB_DIM = 32
EMB_SCALE = 10.0

N_B = 16384
N_S = 50
LANES = 16
BT = 128
N_BT = N_B // BT
NUM_WORKERS = 32
BT_PER_W = N_BT // NUM_WORKERS
XBLK = BT * N_S
NGRP = BT // LANES
SB = 5
GB = SB * BT
NBATCH = N_S // SB
OUTW = 4 * 8 * BT


def _sc_kernel(x_hbm, table_hbm, out_hbm,
               x_blk, idx_sm, rows_a, rows_b, out_a, out_b,
               sem_ga, sem_gb, sem_oa, sem_ob):
    cid = lax.axis_index("c")
    sid = lax.axis_index("s")
    wid = sid * 2 + cid

    lane = lax.iota(jnp.int32, LANES)
    lane50 = lane * N_S
    lane128 = lane * BT

    def fire(k, rows_v, sem):
        return pltpu.async_copy(
            table_hbm.at[idx_sm.at[pl.ds(k * GB, GB)]], rows_v, sem)

    def wait_g(k, rows_v, sem):
        pltpu.make_async_copy(
            table_hbm.at[idx_sm.at[pl.ds(k * GB, GB)]], rows_v, sem).wait()

    @pl.loop(0, BT_PER_W)
    def _bt(j):
        tc = wid * BT_PER_W + j
        pltpu.sync_copy(x_hbm.at[pl.ds(tc * XBLK, XBLK)], x_blk)

        @pl.loop(0, N_S)
        def _ex(s):
            for g in range(NGRP):
                xv = plsc.load_gather(x_blk, [lane50 + (g * LANES * N_S + s)])
                idx_sm[pl.ds(s * BT + g * LANES, LANES)] = xv

        def drain_out(out_t, sem_o):
            for tr in range(4):
                pltpu.make_async_copy(
                    out_t.at[pl.ds(tr * 1024, 1024)],
                    out_hbm.at[0, tr, tc], sem_o).wait()

        def process(k, q, rows_v, sem_g, out_t, sem_o):
            wait_g(k, rows_v, sem_g)

            @pl.loop(0, SB)
            def _s(ss):
                @pl.when((q > 0) | (ss > 0))
                def _dr():
                    drain_out(out_t, sem_o)

                roff = ss * BT

                @plsc.parallel_loop(0, BT, unroll=8)
                def _b(b):
                    for h in range(EMB_DIM // LANES):
                        v = rows_v[roff + b, pl.ds(h * LANES, LANES)]
                        plsc.store_scatter(
                            out_t, [lane128 + (h * LANES * BT + b)],
                            v * EMB_SCALE)

                s_glob = k * SB + ss
                for tr in range(4):
                    pltpu.async_copy(
                        out_t.at[pl.ds(tr * 1024, 1024)],
                        out_hbm.at[s_glob, tr, tc], sem_o)

        fire(0, rows_a, sem_ga)

        @pl.loop(0, NBATCH // 2)
        def _q(q):
            k0 = 2 * q
            fire(k0 + 1, rows_b, sem_gb)
            process(k0, q, rows_a, sem_ga, out_a, sem_oa)

            @pl.when(k0 + 2 < NBATCH)
            def _nx():
                fire(k0 + 2, rows_a, sem_ga)

            process(k0 + 1, q, rows_b, sem_gb, out_b, sem_ob)

        drain_out(out_a, sem_oa)
        drain_out(out_b, sem_ob)


@jax.jit
def _scaled_embedding(x2d, table):
    mesh = plsc.VectorSubcoreMesh(core_axis_name="c", subcore_axis_name="s")
    out6 = pl.kernel(
        _sc_kernel,
        out_type=jax.ShapeDtypeStruct((N_S, 4, N_BT, 8 * BT), jnp.float32),
        mesh=mesh,
        scratch_types=[
            pltpu.VMEM((XBLK,), jnp.int32),
            pltpu.VMEM((XBLK,), jnp.int32),
            pltpu.VMEM((GB, EMB_DIM), jnp.float32),
            pltpu.VMEM((GB, EMB_DIM), jnp.float32),
            pltpu.VMEM((OUTW,), jnp.float32),
            pltpu.VMEM((OUTW,), jnp.float32),
            pltpu.SemaphoreType.DMA,
            pltpu.SemaphoreType.DMA,
            pltpu.SemaphoreType.DMA,
            pltpu.SemaphoreType.DMA,
        ],
        compiler_params=pltpu.CompilerParams(
            use_tc_tiling_on_sc=False, needs_layout_passes=False
        ),
    )(x2d.reshape(-1), table)
    out5 = out6.reshape(N_S, 4, N_BT, 8, BT)
    return out5.transpose((2, 4, 0, 1, 3)).reshape(N_B, N_S, EMB_DIM)


def kernel(x, table):
    return _scaled_embedding(x, table)


# speedup vs baseline: 1.3614x; 1.0062x over previous
import jax
import jax.numpy as jnp
from jax import lax
from jax.experimental import pallas as pl
from jax.experimental.pallas import tpu as pltpu
from jax.experimental.pallas import tpu_sc as plsc

N_EMB = 1000000
EMB_DIM = 32
EMB_SCALE = 10.0

N_B = 16384
N_S = 50
LANES = 16
BT = 128
N_BT = N_B // BT
NUM_WORKERS = 32
BT_PER_W = N_BT // NUM_WORKERS
XBLK = BT * N_S
NGRP = BT // LANES
SB = 5
GB = SB * BT
NBATCH = N_S // SB
OUTW = 4 * 8 * BT


def _sc_kernel(x_hbm, table_hbm, out_hbm,
               x_blk, idx_sm, rows_a, rows_b, out_a, out_b,
               sem_ga, sem_gb, sem_oa, sem_ob):
    cid = lax.axis_index("c")
    sid = lax.axis_index("s")
    wid = sid * 2 + cid

    lane = lax.iota(jnp.int32, LANES)
    lane50 = lane * N_S
    lane128 = lane * BT

    def fire(k, rows_v, sem):
        return pltpu.async_copy(
            table_hbm.at[idx_sm.at[pl.ds(k * GB, GB)]], rows_v, sem)

    def wait_g(k, rows_v, sem):
        pltpu.make_async_copy(
            table_hbm.at[idx_sm.at[pl.ds(k * GB, GB)]], rows_v, sem).wait()

    @pl.loop(0, BT_PER_W)
    def _bt(j):
        tc = wid * BT_PER_W + j
        pltpu.sync_copy(x_hbm.at[pl.ds(tc * XBLK, XBLK)], x_blk)

        @plsc.parallel_loop(0, N_S, unroll=2)
        def _ex(s):
            for g in range(NGRP):
                xv = plsc.load_gather(x_blk, [lane50 + (g * LANES * N_S + s)])
                idx_sm[pl.ds(s * BT + g * LANES, LANES)] = xv

        def drain_out(out_t, sem_o):
            for tr in range(4):
                pltpu.make_async_copy(
                    out_t.at[pl.ds(tr * 1024, 1024)],
                    out_hbm.at[0, tr, tc], sem_o).wait()

        def process(k, q, rows_v, sem_g, out_t, sem_o):
            wait_g(k, rows_v, sem_g)

            @pl.loop(0, SB)
            def _s(ss):
                @pl.when((q > 0) | (ss > 0))
                def _dr():
                    drain_out(out_t, sem_o)

                roff = ss * BT

                @plsc.parallel_loop(0, BT, unroll=8)
                def _b(b):
                    for h in range(EMB_DIM // LANES):
                        v = rows_v[roff + b, pl.ds(h * LANES, LANES)]
                        plsc.store_scatter(
                            out_t, [lane128 + (h * LANES * BT + b)],
                            v * EMB_SCALE)

                s_glob = k * SB + ss
                for tr in range(4):
                    pltpu.async_copy(
                        out_t.at[pl.ds(tr * 1024, 1024)],
                        out_hbm.at[s_glob, tr, tc], sem_o)

        fire(0, rows_a, sem_ga)

        @pl.loop(0, NBATCH // 2)
        def _q(q):
            k0 = 2 * q
            fire(k0 + 1, rows_b, sem_gb)
            process(k0, q, rows_a, sem_ga, out_a, sem_oa)

            @pl.when(k0 + 2 < NBATCH)
            def _nx():
                fire(k0 + 2, rows_a, sem_ga)

            process(k0 + 1, q, rows_b, sem_gb, out_b, sem_ob)

        drain_out(out_a, sem_oa)
        drain_out(out_b, sem_ob)


@jax.jit
def _scaled_embedding(x2d, table):
    mesh = plsc.VectorSubcoreMesh(core_axis_name="c", subcore_axis_name="s")
    out6 = pl.kernel(
        _sc_kernel,
        out_type=jax.ShapeDtypeStruct((N_S, 4, N_BT, 8 * BT), jnp.float32),
        mesh=mesh,
        scratch_types=[
            pltpu.VMEM((XBLK,), jnp.int32),
            pltpu.VMEM((XBLK,), jnp.int32),
            pltpu.VMEM((GB, EMB_DIM), jnp.float32),
            pltpu.VMEM((GB, EMB_DIM), jnp.float32),
            pltpu.VMEM((OUTW,), jnp.float32),
            pltpu.VMEM((OUTW,), jnp.float32),
            pltpu.SemaphoreType.DMA,
            pltpu.SemaphoreType.DMA,
            pltpu.SemaphoreType.DMA,
            pltpu.SemaphoreType.DMA,
        ],
        compiler_params=pltpu.CompilerParams(
            use_tc_tiling_on_sc=False, needs_layout_passes=False
        ),
    )(x2d.reshape(-1), table)
    out5 = out6.reshape(N_S, 4, N_BT, 8, BT)
    return out5.transpose((2, 4, 0, 1, 3)).reshape(N_B, N_S, EMB_DIM)


def kernel(x, table):
    return _scaled_embedding(x, table)
